# baseline (speedup 1.0000x reference)
import functools
import jax
import jax.numpy as jnp
from jax import lax
from jax.experimental import pallas as pl
from jax.experimental.pallas import tpu as pltpu
from jax.experimental.pallas import tpu_sc as plsc

E = 8
TOPK = 2
H = 1024
I = 2048
T = 2048
B = 256
NPAD = 5888
NB = NPAD // B
NTILE = 32
RPT = NPAD // NTILE


def _routing(router_logits):
    probs = jax.nn.softmax(router_logits.astype(jnp.float32), axis=-1)
    topw, topi = jax.lax.top_k(probs, TOPK)
    topw = topw / jnp.sum(topw, axis=-1, keepdims=True)
    return topw, topi.astype(jnp.int32)


def _dispatch(topi):
    N = T * TOPK
    flat_e = topi.reshape(-1)
    perm = jnp.argsort(flat_e, stable=True)
    sorted_e = flat_e[perm]
    counts = jnp.bincount(flat_e, length=E)
    offs = jnp.concatenate([jnp.zeros((1,), jnp.int32),
                            jnp.cumsum(counts)[:-1].astype(jnp.int32)])
    padded = ((counts + B - 1) // B) * B
    bounds = jnp.cumsum(padded).astype(jnp.int32)
    poffs = jnp.concatenate([jnp.zeros((1,), jnp.int32), bounds[:-1]])
    dest = (jnp.arange(N, dtype=jnp.int32) - offs[sorted_e] + poffs[sorted_e])
    order = jnp.zeros((NPAD,), jnp.int32).at[dest].set(
        (perm // TOPK).astype(jnp.int32))
    pos = jnp.zeros((N,), jnp.int32).at[perm].set(dest).reshape(T, TOPK)
    bstart = jnp.arange(32, dtype=jnp.int32) * B
    eb = jnp.minimum((bstart[:, None] >= bounds[None, :]).sum(-1), E - 1)
    num_active = (bounds[-1] // B).astype(jnp.int32)
    block_expert = eb.astype(jnp.int32).at[31].set(num_active)
    return order, pos, block_expert


_GCHUNKS = ((0, 24), (24, 24), (48, 24), (72, 24),
            (96, 24), (120, 24), (144, 24), (168, 16))


def _gather_rows(x, order):
    mesh = plsc.VectorSubcoreMesh(core_axis_name="c", subcore_axis_name="s")

    @functools.partial(
        pl.kernel, mesh=mesh,
        out_type=jax.ShapeDtypeStruct((NPAD, H), jnp.float32),
        scratch_types=(
            [pltpu.VMEM((RPT,), jnp.int32)]
            + [pltpu.VMEM((24, H), jnp.float32) for _ in range(4)]
            + [pltpu.SemaphoreType.DMA for _ in range(4)]
        ),
    )
    def k(x_hbm, order_hbm, xs_hbm, idx_v, b0, b1, b2, b3, s0, s1, s2, s3):
        wid = lax.axis_index("s") * 2 + lax.axis_index("c")
        base = wid * RPT
        pltpu.sync_copy(order_hbm.at[pl.ds(base, RPT)], idx_v)
        bufs = (b0, b1, b2, b3)
        sems = (s0, s1, s2, s3)
        nc = len(_GCHUNKS)
        cps = [None] * nc

        def start(j):
            off, sz = _GCHUNKS[j]
            cps[j] = pltpu.async_copy(
                x_hbm.at[idx_v.at[pl.ds(off, sz)]],
                bufs[j % 4].at[pl.ds(0, sz)], sems[j % 4])

        def finish(j):
            off, sz = _GCHUNKS[j]
            cps[j].wait()
            pltpu.sync_copy(bufs[j % 4].at[pl.ds(0, sz)],
                            xs_hbm.at[pl.ds(base + off, sz)])

        for j in range(4):
            start(j)
        for j in range(4, nc):
            finish(j - 4)
            start(j)
        for j in range(nc - 4, nc):
            finish(j)

    return k(x, order)


def _ffn_body(be_ref, xs_ref, gs_ref, w13_ref, w2_ref, y_ref):
    b = pl.program_id(0)

    @pl.when(b < be_ref[31])
    def _():
        xb = xs_ref[...]
        h = jax.lax.dot_general(xb, w13_ref[0], (((1,), (1,)), ((), ())),
                                preferred_element_type=jnp.float32)
        g = h[:, :I]
        u = h[:, I:]
        act = g * jax.nn.sigmoid(g) * u
        y = jax.lax.dot_general(act, w2_ref[0], (((1,), (1,)), ((), ())),
                                preferred_element_type=jnp.float32)
        y_ref[...] = y * gs_ref[...]

    @pl.when(b >= be_ref[31])
    def _():
        y_ref[...] = jnp.zeros((B, H), jnp.float32)


def _ffn_tc(block_expert, xs, gsort, w13_weight, w2_weight, interpret=False):
    grid_spec = pltpu.PrefetchScalarGridSpec(
        num_scalar_prefetch=1,
        grid=(NB,),
        in_specs=[
            pl.BlockSpec((B, H), lambda b, be: (b, 0)),
            pl.BlockSpec((B, 1), lambda b, be: (b, 0)),
            pl.BlockSpec((1, 2 * I, H), lambda b, be: (be[b], 0, 0)),
            pl.BlockSpec((1, H, I), lambda b, be: (be[b], 0, 0)),
        ],
        out_specs=pl.BlockSpec((B, H), lambda b, be: (b, 0)),
    )
    return pl.pallas_call(
        _ffn_body,
        grid_spec=grid_spec,
        out_shape=jax.ShapeDtypeStruct((NPAD, H), jnp.float32),
        compiler_params=pltpu.CompilerParams(
            dimension_semantics=("arbitrary",),
            vmem_limit_bytes=128 * 1024 * 1024,
        ),
        interpret=interpret,
    )(block_expert, xs, gsort.reshape(NPAD, 1), w13_weight, w2_weight)


TPT = T // NTILE


def _routing_sc(router_logits):
    ltflat = router_logits.astype(jnp.float32).T.reshape(-1)
    mesh = plsc.VectorSubcoreMesh(core_axis_name="c", subcore_axis_name="s")

    @functools.partial(
        pl.kernel, mesh=mesh,
        out_type=(jax.ShapeDtypeStruct((TOPK * T,), jnp.float32),
                  jax.ShapeDtypeStruct((TOPK * T,), jnp.int32)),
        scratch_types=[
            pltpu.VMEM((E * TPT,), jnp.float32),
            pltpu.VMEM((TPT,), jnp.float32),
            pltpu.VMEM((TPT,), jnp.float32),
            pltpu.VMEM((TPT,), jnp.int32),
            pltpu.VMEM((TPT,), jnp.int32),
        ],
    )
    def k(lt_hbm, g_hbm, e_hbm, ltv, g0v, g1v, e0v, e1v):
        wid = lax.axis_index("s") * 2 + lax.axis_index("c")
        base = wid * TPT
        for e in range(E):
            pltpu.sync_copy(lt_hbm.at[pl.ds(e * T + base, TPT)],
                            ltv.at[pl.ds(e * TPT, TPT)])
        for g in range(TPT // 16):
            l0 = ltv[pl.ds(g * 16, 16)]
            m0 = l0
            i0 = jnp.zeros((16,), jnp.int32)
            m1 = jnp.full((16,), -3.0e38, jnp.float32)
            i1 = jnp.zeros((16,), jnp.int32)
            for e in range(1, E):
                le = ltv[pl.ds(e * TPT + g * 16, 16)]
                ev = jnp.full((16,), e, jnp.int32)
                gt0 = le > m0
                gt1 = le > m1
                i1 = jnp.where(gt0, i0, jnp.where(gt1, ev, i1))
                m1 = jnp.where(gt0, m0, jnp.where(gt1, le, m1))
                i0 = jnp.where(gt0, ev, i0)
                m0 = jnp.where(gt0, le, m0)
            t = jnp.exp(m1 - m0)
            d = jnp.float32(1.0) + t
            g0v[pl.ds(g * 16, 16)] = jnp.float32(1.0) / d
            g1v[pl.ds(g * 16, 16)] = t / d
            e0v[pl.ds(g * 16, 16)] = i0
            e1v[pl.ds(g * 16, 16)] = i1
        pltpu.sync_copy(g0v, g_hbm.at[pl.ds(base, TPT)])
        pltpu.sync_copy(g1v, g_hbm.at[pl.ds(T + base, TPT)])
        pltpu.sync_copy(e0v, e_hbm.at[pl.ds(base, TPT)])
        pltpu.sync_copy(e1v, e_hbm.at[pl.ds(T + base, TPT)])

    gflat, eflat = k(ltflat)
    g2 = gflat.reshape(TOPK, T)
    e2 = eflat.reshape(TOPK, T)
    return (g2[0], g2[1]), (e2[0], e2[1])


def _combine(y, pos):
    return y[pos[0]] + y[pos[1]]


def kernel(x, router_logits, w13_weight, w2_weight):
    gates2, eids2 = _routing_sc(router_logits)
    gates = jnp.stack([gates2[0], gates2[1]], axis=1)
    topi = jnp.stack([eids2[0], eids2[1]], axis=1)
    order, pos, block_expert = _dispatch(topi)
    gsort = jnp.zeros((NPAD,), jnp.float32).at[pos.reshape(-1)].set(
        gates.reshape(-1))
    xs = _gather_rows(x, order)
    y = _ffn_tc(block_expert, xs, gsort, w13_weight, w2_weight)
    return _combine(y, (pos[:, 0], pos[:, 1]))
